# Initial kernel scaffold; baseline (speedup 1.0000x reference)
import jax
import jax.numpy as jnp
from jax.experimental import pallas as pl


def kernel(x1_features, x1_pos, x1_batch, x2_features, x2_pos, x2_batch, W1, b1, W2, b2):
    raise NotImplementedError("write your pallas kernel here")


# speedup vs baseline: 7.0949x; 7.0949x over previous
import functools

import jax
import jax.numpy as jnp
from jax import lax
from jax.experimental import pallas as pl
from jax.experimental.pallas import tpu as pltpu
from jax.experimental.pallas import tpu_sc as plsc

_N1 = 16384
_N2 = 16384
_D = 128
_NB = 16
_HID = 128

_BM = 2048
_BQ = 256
_BK = 512
_NQB = _N2 // _BQ


def _u_body(xf_ref, xp_ref, w1a_ref, w1b_ref, b1_ref, u_ref):
    u = jnp.dot(xf_ref[...], w1a_ref[...], preferred_element_type=jnp.float32)
    u += jnp.dot(xp_ref[...], w1b_ref[...], preferred_element_type=jnp.float32)
    u_ref[...] = u + b1_ref[...]


def _knn_body(bounds_ref, x2p_ref, x2b_ref, x1pt_ref, x1b_ref, col_ref):
    q = pl.program_id(0)
    lo = bounds_ref[q, 0]
    hi = bounds_ref[q, 1]
    kb0 = lo // _BK
    kb1 = (hi + _BK - 1) // _BK
    qx = x2p_ref[:, 0:1]
    qy = x2p_ref[:, 1:2]
    qz = x2p_ref[:, 2:3]
    qb = x2b_ref[...]
    inf = jnp.float32(jnp.inf)

    def tile(kb, carry):
        bd, bi = carry
        off = kb * _BK
        px = x1pt_ref[0:1, pl.ds(off, _BK)]
        py = x1pt_ref[1:2, pl.ds(off, _BK)]
        pz = x1pt_ref[2:3, pl.ds(off, _BK)]
        tb = x1b_ref[0:1, pl.ds(off, _BK)]
        d = (qx - px) ** 2 + (qy - py) ** 2 + (qz - pz) ** 2
        d = jnp.where(qb == tb, d, inf)
        tmin = jnp.min(d, axis=1, keepdims=True)
        lane = lax.broadcasted_iota(jnp.int32, (_BQ, _BK), 1) + off
        cand = jnp.where(d == tmin, lane, jnp.int32(2 ** 30))
        targ = jnp.min(cand, axis=1, keepdims=True)
        upd = tmin < bd
        return jnp.where(upd, tmin, bd), jnp.where(upd, targ, bi)

    bd0 = jnp.full((_BQ, 1), inf, jnp.float32)
    bi0 = jnp.zeros((_BQ, 1), jnp.int32)
    _, bi = lax.fori_loop(kb0, kb1, tile, (bd0, bi0))
    col_ref[...] = bi.reshape(1, _BQ, 1)


def _mlp_body(g_ref, x2p_ref, w1b_ref, w2_ref, b2_ref, o_ref):
    v = jnp.dot(x2p_ref[...], w1b_ref[...], preferred_element_type=jnp.float32)
    h1 = jnp.maximum(g_ref[...] - v, 0.0)
    h2 = jnp.dot(h1, w2_ref[...], preferred_element_type=jnp.float32) + b2_ref[...]
    o_ref[...] = jnp.maximum(h2, 0.0)


def kernel(x1_features, x1_pos, x1_batch, x2_features, x2_pos, x2_batch,
           W1, b1, W2, b2):
    x1p8 = jnp.pad(x1_pos, ((0, 0), (0, 5)))
    x2p8 = jnp.pad(x2_pos, ((0, 0), (0, 5)))
    w1a = W1[:_D]
    w1b8 = jnp.pad(W1[_D:], ((0, 5), (0, 0)))
    b1r = b1.reshape(1, _HID)
    b2r = b2.reshape(1, _HID)
    x1pt = x1p8.T
    x1b2 = x1_batch.reshape(1, _N1).astype(jnp.int32)
    x2b2 = x2_batch.reshape(_N2, 1).astype(jnp.int32)

    bids = jnp.arange(_NB, dtype=x1_batch.dtype)
    starts = jnp.searchsorted(x1_batch, bids, side='left').astype(jnp.int32)
    ends = jnp.searchsorted(x1_batch, bids, side='right').astype(jnp.int32)
    blo = x2_batch[0::_BQ]
    bhi = x2_batch[_BQ - 1::_BQ]
    bounds = jnp.stack([starts[blo], ends[bhi]], axis=1).astype(jnp.int32)

    u = pl.pallas_call(
        _u_body,
        grid=(_N1 // _BM,),
        in_specs=[
            pl.BlockSpec((_BM, _D), lambda i: (i, 0)),
            pl.BlockSpec((_BM, 8), lambda i: (i, 0)),
            pl.BlockSpec((_D, _HID), lambda i: (0, 0)),
            pl.BlockSpec((8, _HID), lambda i: (0, 0)),
            pl.BlockSpec((1, _HID), lambda i: (0, 0)),
        ],
        out_specs=pl.BlockSpec((_BM, _HID), lambda i: (i, 0)),
        out_shape=jax.ShapeDtypeStruct((_N1, _HID), jnp.float32),
    )(x1_features, x1p8, w1a, w1b8, b1r)

    col3 = pl.pallas_call(
        _knn_body,
        grid_spec=pltpu.PrefetchScalarGridSpec(
            num_scalar_prefetch=1,
            grid=(_NQB,),
            in_specs=[
                pl.BlockSpec((_BQ, 8), lambda q, b: (q, 0)),
                pl.BlockSpec((_BQ, 1), lambda q, b: (q, 0)),
                pl.BlockSpec((8, _N1), lambda q, b: (0, 0)),
                pl.BlockSpec((1, _N1), lambda q, b: (0, 0)),
            ],
            out_specs=pl.BlockSpec((1, _BQ, 1), lambda q, b: (q, 0, 0)),
        ),
        out_shape=jax.ShapeDtypeStruct((_NQB, _BQ, 1), jnp.int32),
    )(bounds, x2p8, x2b2, x1pt, x1b2)
    col = col3.reshape(_N2)

    info = plsc.get_sparse_core_info()
    nw = info.num_cores * info.num_subcores
    bpw = _N2 // nw
    nch = bpw // 128
    col3d = col.reshape(nw, nch, 128)
    mesh = plsc.VectorSubcoreMesh(core_axis_name="c", subcore_axis_name="s")

    @functools.partial(
        pl.kernel,
        out_type=jax.ShapeDtypeStruct((_N2, _HID), jnp.float32),
        mesh=mesh,
        scratch_types=[
            pltpu.VMEM((nch, 128), jnp.int32),
            pltpu.VMEM((bpw, _HID), jnp.float32),
            pltpu.SemaphoreType.DMA,
        ],
    )
    def _sc_gather(u_hbm, idx_hbm, out_hbm, idx_v, rows_v, sem):
        w = lax.axis_index("s") * info.num_cores + lax.axis_index("c")
        pltpu.sync_copy(idx_hbm.at[w], idx_v)
        cps = [
            pltpu.async_copy(u_hbm.at[idx_v.at[j]],
                             rows_v.at[pl.ds(j * 128, 128)], sem)
            for j in range(nch)
        ]
        for cp in cps:
            cp.wait()
        pltpu.sync_copy(rows_v, out_hbm.at[pl.ds(w * bpw, bpw)])

    g = _sc_gather(u, col3d)

    out = pl.pallas_call(
        _mlp_body,
        grid=(_N2 // _BM,),
        in_specs=[
            pl.BlockSpec((_BM, _HID), lambda i: (i, 0)),
            pl.BlockSpec((_BM, 8), lambda i: (i, 0)),
            pl.BlockSpec((8, _HID), lambda i: (0, 0)),
            pl.BlockSpec((_HID, _HID), lambda i: (0, 0)),
            pl.BlockSpec((1, _HID), lambda i: (0, 0)),
        ],
        out_specs=pl.BlockSpec((_BM, _HID), lambda i: (i, 0)),
        out_shape=jax.ShapeDtypeStruct((_N2, _HID), jnp.float32),
    )(g, x2p8, w1b8, W2, b2r)

    return (out, x2_pos, x2_batch)
